# baseline (speedup 1.0000x reference)
import functools

import jax
import jax.numpy as jnp
from jax import lax
from jax.experimental import pallas as pl
from jax.experimental.pallas import tpu as pltpu
from jax.experimental.pallas import tpu_sc as plsc

N_ITEMS = 16384
D = 64
NF = 5
N_STYLES = 1000
N_BREWERS = 100000
EPS9 = 9e-5

NC = 2
NS = 16
NW = NC * NS
CHUNK = 2048
NCHUNK = N_ITEMS // CHUNK
BPW = N_ITEMS // NW
NG = BPW // 16

_MAGIC = 0x5F3759DF


def _gather_body(bt_h, st_h, bid_h, sid_h, h1_h,
                 brow_v, srow_v, bid_v, sid_v, out_v, sem):
    wid = lax.axis_index("s") * NC + lax.axis_index("c")
    for p in range(2):
        dim = p * NW + wid
        pltpu.sync_copy(bt_h.at[dim], brow_v)
        pltpu.sync_copy(st_h.at[dim], srow_v)

        def chunk_body(ci, carry):
            base = ci * CHUNK
            pltpu.sync_copy(bid_h.at[pl.ds(base, CHUNK)], bid_v)
            pltpu.sync_copy(sid_h.at[pl.ds(base, CHUNK)], sid_v)
            for v in range(CHUNK // 16):
                bi = bid_v[pl.ds(16 * v, 16)]
                si = sid_v[pl.ds(16 * v, 16)]
                g = plsc.load_gather(brow_v, [bi]) + plsc.load_gather(srow_v, [si])
                out_v[pl.ds(16 * v, 16)] = g
            pltpu.sync_copy(out_v, h1_h.at[dim, pl.ds(base, CHUNK)])
            return carry

        lax.fori_loop(0, NCHUNK, chunk_body, 0)


def _ln_body(h1_h, ct_h, wt_h, b_h, g_h, be_h, out_h,
             blk_v, cf_v, wt_v, b_v, g_v, be_v, sem):
    wid = lax.axis_index("s") * NC + lax.axis_index("c")
    base = wid * BPW
    pltpu.sync_copy(h1_h.at[pl.ds(0, D), pl.ds(base, BPW)], blk_v)
    for k in range(NF):
        pltpu.sync_copy(ct_h.at[pl.ds(k, 1), pl.ds(base, BPW)], cf_v.at[pl.ds(k, 1)])
    pltpu.sync_copy(wt_h, wt_v)
    pltpu.sync_copy(b_h, b_v)
    pltpu.sync_copy(g_h, g_v)
    pltpu.sync_copy(be_h, be_v)

    wvec = [[wt_v[k, pl.ds(16 * j, 16)] for j in range(4)] for k in range(NF)]
    bvec = [b_v[pl.ds(16 * j, 16)] for j in range(4)]
    gvec = [g_v[pl.ds(16 * j, 16)] for j in range(4)]
    bevec = [be_v[pl.ds(16 * j, 16)] for j in range(4)]
    inv_d = jnp.float32(1.0 / D)

    def group_body(g, carry):
        cols = [g * 32, g * 32 + 16]
        cf = [[cf_v[k, pl.ds(c, 16)] for k in range(NF)] for c in cols]
        s1 = [jnp.zeros((16,), jnp.float32) for _ in cols]
        s2 = [jnp.zeros((16,), jnp.float32) for _ in cols]
        for d in range(D):
            w_d = [wvec[k][d // 16][d % 16] for k in range(NF)]
            b_d = bvec[d // 16][d % 16]
            for t, c in enumerate(cols):
                h = blk_v[d, pl.ds(c, 16)] + b_d
                for k in range(NF):
                    h = h + w_d[k] * cf[t][k]
                blk_v[d, pl.ds(c, 16)] = h
                s1[t] = s1[t] + h
                s2[t] = s2[t] + h * h
        inv = []
        mean = []
        for t in range(2):
            m = s1[t] * inv_d
            var = s2[t] * inv_d - m * m + jnp.float32(EPS9)
            iv = _MAGIC - lax.shift_right_arithmetic(plsc.bitcast(var, jnp.int32), 1)
            y = plsc.bitcast(iv, jnp.float32)
            for _ in range(3):
                y = y * (jnp.float32(1.5) - jnp.float32(0.5) * var * y * y)
            inv.append(y)
            mean.append(m)
        for d in range(D):
            g_d = gvec[d // 16][d % 16]
            be_d = bevec[d // 16][d % 16]
            for t, c in enumerate(cols):
                h = blk_v[d, pl.ds(c, 16)]
                blk_v[d, pl.ds(c, 16)] = (h - mean[t]) * inv[t] * g_d + be_d
        return carry

    lax.fori_loop(0, NG // 2, group_body, 0)
    pltpu.sync_copy(blk_v, out_h.at[pl.ds(0, D), pl.ds(base, BPW)])


def kernel(style_ids, brewer_ids, cont_feats, style_table, brewer_table, W, b, gamma, beta):
    bt = brewer_table.T
    st = style_table.T
    ct = cont_feats.T
    wt = W.T

    mesh = plsc.VectorSubcoreMesh(core_axis_name="c", subcore_axis_name="s")
    params = pltpu.CompilerParams(needs_layout_passes=False, use_tc_tiling_on_sc=True)

    gather = pl.kernel(
        _gather_body,
        out_type=jax.ShapeDtypeStruct((D, N_ITEMS), jnp.float32),
        mesh=mesh,
        compiler_params=params,
        scratch_types=[
            pltpu.VMEM((N_BREWERS,), jnp.float32),
            pltpu.VMEM((N_STYLES,), jnp.float32),
            pltpu.VMEM((CHUNK,), jnp.int32),
            pltpu.VMEM((CHUNK,), jnp.int32),
            pltpu.VMEM((CHUNK,), jnp.float32),
            pltpu.SemaphoreType.DMA,
        ],
    )
    h1 = gather(bt, st, brewer_ids, style_ids)

    ln = pl.kernel(
        _ln_body,
        out_type=jax.ShapeDtypeStruct((D, N_ITEMS), jnp.float32),
        mesh=mesh,
        compiler_params=params,
        scratch_types=[
            pltpu.VMEM((D, BPW), jnp.float32),
            pltpu.VMEM((NF, BPW), jnp.float32),
            pltpu.VMEM((NF, D), jnp.float32),
            pltpu.VMEM((D,), jnp.float32),
            pltpu.VMEM((D,), jnp.float32),
            pltpu.VMEM((D,), jnp.float32),
            pltpu.SemaphoreType.DMA,
        ],
    )
    out_t = ln(h1, ct, wt, b, gamma, beta)
    return out_t.T


# speedup vs baseline: 2.1657x; 2.1657x over previous
import functools

import jax
import jax.numpy as jnp
from jax import lax
from jax.experimental import pallas as pl
from jax.experimental.pallas import tpu as pltpu
from jax.experimental.pallas import tpu_sc as plsc

N_ITEMS = 16384
D = 64
NF = 5
N_STYLES = 1000
N_BREWERS = 100000
EPS9 = 9e-5

NC = 2
NS = 16
NW = NC * NS
CHUNK = 2048
NCHUNK = N_ITEMS // CHUNK
BLK = 512
NBLK = N_ITEMS // BLK


def _gather_body(bt_h, st_h, bid_h, sid_h, h1_h,
                 brow_v, srow_v, bid_v, sid_v, out_v,
                 sem_row, sem_ids, sem_out):
    wid = lax.axis_index("s") * NC + lax.axis_index("c")

    def fetch_ids(ci, buf):
        base = ci * CHUNK
        pltpu.async_copy(bid_h.at[pl.ds(base, CHUNK)], bid_v.at[buf], sem_ids)
        pltpu.async_copy(sid_h.at[pl.ds(base, CHUNK)], sid_v.at[buf], sem_ids)

    def drain_ids(ci, buf):
        pltpu.make_async_copy(bid_h.at[pl.ds(ci * CHUNK, CHUNK)], bid_v.at[buf], sem_ids).wait()
        pltpu.make_async_copy(sid_h.at[pl.ds(ci * CHUNK, CHUNK)], sid_v.at[buf], sem_ids).wait()

    for p in range(2):
        dim = p * NW + wid
        rcp = pltpu.async_copy(bt_h.at[dim], brow_v, sem_row)
        scp = pltpu.async_copy(st_h.at[dim], srow_v, sem_row)
        if p == 0:
            fetch_ids(0, 0)
        rcp.wait()
        scp.wait()

        def chunk_body(ci, carry):
            buf = lax.rem(ci, 2)
            drain_ids(ci, buf)

            @pl.when(ci + 1 < NCHUNK)
            def _():
                fetch_ids(ci + 1, 1 - buf)

            @pl.when(ci >= 2)
            def _():
                pltpu.make_async_copy(
                    out_v.at[buf], h1_h.at[dim, pl.ds((ci - 2) * CHUNK, CHUNK)],
                    sem_out).wait()

            for v0 in range(0, CHUNK // 16, 8):
                bis = [bid_v[buf, pl.ds(16 * (v0 + u), 16)] for u in range(8)]
                sis = [sid_v[buf, pl.ds(16 * (v0 + u), 16)] for u in range(8)]
                gb = [plsc.load_gather(brow_v, [bi]) for bi in bis]
                gs = [plsc.load_gather(srow_v, [si]) for si in sis]
                for u in range(8):
                    out_v[buf, pl.ds(16 * (v0 + u), 16)] = gb[u] + gs[u]
            pltpu.async_copy(out_v.at[buf], h1_h.at[dim, pl.ds(ci * CHUNK, CHUNK)], sem_out)
            return carry

        lax.fori_loop(0, NCHUNK, chunk_body, 0)
        for ci in range(NCHUNK - 2, NCHUNK):
            pltpu.make_async_copy(
                out_v.at[ci % 2], h1_h.at[dim, pl.ds(ci * CHUNK, CHUNK)],
                sem_out).wait()
        if p == 0:
            fetch_ids(0, 0)


def _ln_tc_body(h1_ref, ct_ref, w_ref, b_ref, g_ref, be_ref, out_ref):
    c = jax.lax.dot_general(
        w_ref[...], ct_ref[...], (((1,), (0,)), ((), ())),
        preferred_element_type=jnp.float32)
    h = h1_ref[...] + c + b_ref[...]
    mean = jnp.mean(h, axis=0, keepdims=True)
    var = jnp.mean(h * h, axis=0, keepdims=True) - mean * mean
    inv = jax.lax.rsqrt(var + EPS9)
    out_ref[...] = (h - mean) * inv * g_ref[...] + be_ref[...]


def kernel(style_ids, brewer_ids, cont_feats, style_table, brewer_table, W, b, gamma, beta):
    bt = brewer_table.T
    st = style_table.T
    ct = cont_feats.T

    mesh = plsc.VectorSubcoreMesh(core_axis_name="c", subcore_axis_name="s")
    params = pltpu.CompilerParams(needs_layout_passes=False, use_tc_tiling_on_sc=True)

    gather = pl.kernel(
        _gather_body,
        out_type=jax.ShapeDtypeStruct((D, N_ITEMS), jnp.float32),
        mesh=mesh,
        compiler_params=params,
        scratch_types=[
            pltpu.VMEM((N_BREWERS,), jnp.float32),
            pltpu.VMEM((N_STYLES,), jnp.float32),
            pltpu.VMEM((2, CHUNK), jnp.int32),
            pltpu.VMEM((2, CHUNK), jnp.int32),
            pltpu.VMEM((2, CHUNK), jnp.float32),
            pltpu.SemaphoreType.DMA,
            pltpu.SemaphoreType.DMA,
            pltpu.SemaphoreType.DMA,
        ],
    )
    h1 = gather(bt, st, brewer_ids, style_ids)

    out_t = pl.pallas_call(
        _ln_tc_body,
        out_shape=jax.ShapeDtypeStruct((D, N_ITEMS), jnp.float32),
        grid=(NBLK,),
        in_specs=[
            pl.BlockSpec((D, BLK), lambda i: (0, i)),
            pl.BlockSpec((NF, BLK), lambda i: (0, i)),
            pl.BlockSpec((D, NF), lambda i: (0, 0)),
            pl.BlockSpec((D, 1), lambda i: (0, 0)),
            pl.BlockSpec((D, 1), lambda i: (0, 0)),
            pl.BlockSpec((D, 1), lambda i: (0, 0)),
        ],
        out_specs=pl.BlockSpec((D, BLK), lambda i: (0, i)),
    )(h1, ct, W, b.reshape(D, 1), gamma.reshape(D, 1), beta.reshape(D, 1))
    return out_t.T


# speedup vs baseline: 2.5495x; 1.1772x over previous
import functools

import jax
import jax.numpy as jnp
from jax import lax
from jax.experimental import pallas as pl
from jax.experimental.pallas import tpu as pltpu
from jax.experimental.pallas import tpu_sc as plsc

N_ITEMS = 16384
D = 64
NF = 5
N_STYLES = 1000
N_BREWERS = 100000
EPS9 = 9e-5

NC = 2
NS = 16
NW = NC * NS
CHUNK = 4096
NCHUNK = N_ITEMS // CHUNK
BLK = 2048
NBLK = N_ITEMS // BLK
ROW_PAD = 100096


def _gather_body(bt_h, st_h, bid_h, sid_h, h1_h,
                 brow_v, srow_v, bid_v, sid_v, out_v,
                 sem_row, sem_ids, sem_out):
    wid = lax.axis_index("s") * NC + lax.axis_index("c")

    def fetch_ids(ci, buf):
        base = ci * CHUNK
        pltpu.async_copy(bid_h.at[pl.ds(base, CHUNK)], bid_v.at[buf], sem_ids)
        pltpu.async_copy(sid_h.at[pl.ds(base, CHUNK)], sid_v.at[buf], sem_ids)

    def drain_ids(ci, buf):
        pltpu.make_async_copy(bid_h.at[pl.ds(ci * CHUNK, CHUNK)], bid_v.at[buf], sem_ids).wait()
        pltpu.make_async_copy(sid_h.at[pl.ds(ci * CHUNK, CHUNK)], sid_v.at[buf], sem_ids).wait()

    for p in range(2):
        dim = p * NW + wid
        rcp = pltpu.async_copy(bt_h.at[dim], brow_v, sem_row)
        scp = pltpu.async_copy(st_h.at[dim], srow_v, sem_row)
        if p == 0:
            fetch_ids(0, 0)
        rcp.wait()
        scp.wait()

        def chunk_body(ci, carry):
            buf = lax.rem(ci, 2)
            drain_ids(ci, buf)

            @pl.when(ci + 1 < NCHUNK)
            def _():
                fetch_ids(ci + 1, 1 - buf)

            @pl.when(ci >= 2)
            def _():
                pltpu.make_async_copy(
                    out_v.at[buf], h1_h.at[dim, pl.ds((ci - 2) * CHUNK, CHUNK)],
                    sem_out).wait()

            for v0 in range(0, CHUNK // 16, 8):
                bis = [bid_v[buf, pl.ds(16 * (v0 + u), 16)] for u in range(8)]
                sis = [sid_v[buf, pl.ds(16 * (v0 + u), 16)] for u in range(8)]
                gb = [plsc.load_gather(brow_v, [bi]) for bi in bis]
                gs = [plsc.load_gather(srow_v, [si]) for si in sis]
                for u in range(8):
                    out_v[buf, pl.ds(16 * (v0 + u), 16)] = gb[u] + gs[u]
            pltpu.async_copy(out_v.at[buf], h1_h.at[dim, pl.ds(ci * CHUNK, CHUNK)], sem_out)
            return carry

        lax.fori_loop(0, NCHUNK, chunk_body, 0)
        for ci in range(NCHUNK - 2, NCHUNK):
            pltpu.make_async_copy(
                out_v.at[ci % 2], h1_h.at[dim, pl.ds(ci * CHUNK, CHUNK)],
                sem_out).wait()
        if p == 0:
            fetch_ids(0, 0)


def _ln_tc_body(h1_ref, ct_ref, wbg_ref, out_ref):
    wbg = wbg_ref[...]
    w = wbg[:, 0:NF]
    b = wbg[:, NF:NF + 1]
    g = wbg[:, NF + 1:NF + 2]
    be = wbg[:, NF + 2:NF + 3]
    c = jax.lax.dot_general(
        w, ct_ref[...], (((1,), (0,)), ((), ())),
        preferred_element_type=jnp.float32)
    h = h1_ref[...] + c + b
    mean = jnp.mean(h, axis=0, keepdims=True)
    var = jnp.mean(h * h, axis=0, keepdims=True) - mean * mean
    inv = jax.lax.rsqrt(var + EPS9)
    out_ref[...] = (h - mean) * inv * g + be


def kernel(style_ids, brewer_ids, cont_feats, style_table, brewer_table, W, b, gamma, beta):
    bt = brewer_table.T
    st = style_table.T
    ct = cont_feats.T

    mesh = plsc.VectorSubcoreMesh(core_axis_name="c", subcore_axis_name="s")
    params = pltpu.CompilerParams(needs_layout_passes=False, use_tc_tiling_on_sc=True)

    gather = pl.kernel(
        _gather_body,
        out_type=jax.ShapeDtypeStruct((D, N_ITEMS), jnp.float32),
        mesh=mesh,
        compiler_params=params,
        scratch_types=[
            pltpu.VMEM((N_BREWERS,), jnp.float32),
            pltpu.VMEM((N_STYLES,), jnp.float32),
            pltpu.VMEM((2, CHUNK), jnp.int32),
            pltpu.VMEM((2, CHUNK), jnp.int32),
            pltpu.VMEM((2, CHUNK), jnp.float32),
            pltpu.SemaphoreType.DMA,
            pltpu.SemaphoreType.DMA,
            pltpu.SemaphoreType.DMA,
        ],
    )
    h1 = gather(bt, st, brewer_ids, style_ids)

    wbg = jnp.concatenate(
        [W, b[:, None], gamma[:, None], beta[:, None]], axis=1)
    out_t = pl.pallas_call(
        _ln_tc_body,
        out_shape=jax.ShapeDtypeStruct((D, N_ITEMS), jnp.float32),
        grid=(NBLK,),
        in_specs=[
            pl.BlockSpec((D, BLK), lambda i: (0, i)),
            pl.BlockSpec((NF, BLK), lambda i: (0, i)),
            pl.BlockSpec((D, NF + 3), lambda i: (0, 0)),
        ],
        out_specs=pl.BlockSpec((D, BLK), lambda i: (0, i)),
    )(h1, ct, wbg)
    return out_t.T
